# Initial kernel scaffold; baseline (speedup 1.0000x reference)
import functools

import jax
import jax.numpy as jnp
from jax import lax
from jax.experimental import pallas as pl
from jax.experimental.pallas import tpu as pltpu
from jax.experimental.pallas import tpu_sc as plsc

N = 10000
NP = 10240
F_IN = 128
HID = 128
LAT = 64
NUM_GRAPHS = 64
NEG_SLOPE = 0.2

E = 320000
ET = E + N
EP = 330240
NW = 32
EPT = EP // NW
NBLK = EPT // 16
CH = NP // 16

_MESH = plsc.VectorSubcoreMesh(
    core_axis_name="c", subcore_axis_name="s", num_cores=2, num_subcores=16)


def _dot(a, b):
  return jax.lax.dot_general(
      a, b, (((a.ndim - 1,), (0,)), ((), ())),
      precision=jax.lax.Precision.HIGHEST,
      preferred_element_type=jnp.float32)


def _tc_prologue_body(x_ref, w_ref, aa_ref, h_ref, al_ref):
  h = _dot(x_ref[...], w_ref[...])
  h_ref[...] = h
  al_ref[...] = _dot(h, aa_ref[...])


def _tc_mid_body(agg_ref, b_ref, w_ref, aa_ref, h_ref, al_ref):
  xin = jax.nn.relu(agg_ref[0] + agg_ref[1] + b_ref[...])
  h = _dot(xin, w_ref[...])
  h_ref[...] = h
  al_ref[...] = _dot(h, aa_ref[...])


def _tc_final_body(agg_ref, b_ref, batch_ref, fcw_ref, fcb_ref, z_ref):
  hf = jax.nn.relu(agg_ref[0] + agg_ref[1] + b_ref[...])
  ids = jnp.broadcast_to(batch_ref[...], (NUM_GRAPHS, NP))
  gids = lax.broadcasted_iota(jnp.int32, (NUM_GRAPHS, NP), 0)
  p = jnp.where(ids == gids, 1.0, 0.0).astype(jnp.float32)
  sums = _dot(p, hf)
  counts = jnp.sum(p, axis=1, keepdims=True)
  pooled = sums / jnp.maximum(counts, 1.0)
  z_ref[...] = _dot(pooled, fcw_ref[...]) + fcb_ref[...]


def _tc_prologue(x, w, aa):
  return pl.pallas_call(
      _tc_prologue_body,
      out_shape=[jax.ShapeDtypeStruct((NP, HID), jnp.float32),
                 jax.ShapeDtypeStruct((NP, 2), jnp.float32)],
  )(x, w, aa)


def _tc_mid(agg, b, w, aa):
  return pl.pallas_call(
      _tc_mid_body,
      out_shape=[jax.ShapeDtypeStruct((NP, HID), jnp.float32),
                 jax.ShapeDtypeStruct((NP, 2), jnp.float32)],
  )(agg, b, w, aa)


def _tc_final(agg, b, batch2d, fcw, fcb):
  return pl.pallas_call(
      _tc_final_body,
      out_shape=jax.ShapeDtypeStruct((NUM_GRAPHS, LAT), jnp.float32),
  )(agg, b, batch2d, fcw, fcb)


def _sc_attn_body(src_h, dst_h, al_h, ex_h, den_h,
                  src_v, dst_v, tbl_v, ex_v, den_v, red_v, tmp_v,
                  den_sh, sem):
  c = lax.axis_index("c")
  s = lax.axis_index("s")
  wid = c * 16 + s
  base = wid * EPT

  pltpu.sync_copy(src_h.at[pl.ds(base, EPT)], src_v)
  pltpu.sync_copy(dst_h.at[pl.ds(base, EPT)], dst_v)
  pltpu.sync_copy(al_h, tbl_v)

  zf16 = jnp.zeros((16,), jnp.float32)

  def zero_body(i, carry):
    den_v[pl.ds(i * 16, 16)] = zf16
    return carry
  lax.fori_loop(0, NP // 16, zero_body, 0)

  zi16 = jnp.zeros((16,), jnp.int32)
  oi16 = jnp.ones((16,), jnp.int32)
  lane = lax.iota(jnp.int32, 16)

  def edge_body(i, carry):
    off = i * 16
    s16 = src_v[pl.ds(off, 16)]
    d16 = dst_v[pl.ds(off, 16)]
    a_s = plsc.load_gather(tbl_v, [s16, zi16])
    a_d = plsc.load_gather(tbl_v, [d16, oi16])
    e = a_s + a_d
    e = jnp.where(e >= 0.0, e, NEG_SLOPE * e)
    ex = jnp.exp(e)
    gid = base + off + lane
    ex = jnp.where(gid < ET, ex, 0.0)
    ex_v[pl.ds(off, 16)] = ex
    plsc.addupdate_scatter(den_v, [d16], ex)
    return carry
  lax.fori_loop(0, NBLK, edge_body, 0)

  pltpu.sync_copy(ex_v, ex_h.at[pl.ds(base, EPT)])

  pltpu.sync_copy(den_v, den_sh.at[s])
  plsc.subcore_barrier()
  cbase = s * CH
  pltpu.sync_copy(den_sh.at[0, pl.ds(cbase, CH)], red_v)
  for t in range(1, 16):
    pltpu.sync_copy(den_sh.at[t, pl.ds(cbase, CH)], tmp_v)

    def add_body(j, carry):
      red_v[pl.ds(j * 16, 16)] = (red_v[pl.ds(j * 16, 16)]
                                  + tmp_v[pl.ds(j * 16, 16)])
      return carry
    lax.fori_loop(0, CH // 16, add_body, 0)
  pltpu.sync_copy(red_v, den_h.at[c, pl.ds(cbase, CH)])


_sc_attn = pl.kernel(
    _sc_attn_body,
    out_type=[jax.ShapeDtypeStruct((EP,), jnp.float32),
              jax.ShapeDtypeStruct((2, NP), jnp.float32)],
    mesh=_MESH,
    scratch_types=[
        pltpu.VMEM((EPT,), jnp.int32),
        pltpu.VMEM((EPT,), jnp.int32),
        pltpu.VMEM((NP, 2), jnp.float32),
        pltpu.VMEM((EPT,), jnp.float32),
        pltpu.VMEM((NP,), jnp.float32),
        pltpu.VMEM((CH,), jnp.float32),
        pltpu.VMEM((CH,), jnp.float32),
        pltpu.VMEM_SHARED((16, NP), jnp.float32),
        pltpu.SemaphoreType.DMA,
    ],
)


def _sc_agg_body(src_h, dst_h, ex_h, den_h, h_h, out_h,
                 src_v, dst_v, ex_v, den_v, tmp_v, rows_v, wrows_v, wb_v,
                 zrow_v, acc_sh, sem):
  c = lax.axis_index("c")
  s = lax.axis_index("s")
  wid = c * 16 + s
  base = wid * EPT

  pltpu.sync_copy(src_h.at[pl.ds(base, EPT)], src_v)
  pltpu.sync_copy(dst_h.at[pl.ds(base, EPT)], dst_v)
  pltpu.sync_copy(ex_h.at[pl.ds(base, EPT)], ex_v)
  pltpu.sync_copy(den_h.at[0], den_v)
  pltpu.sync_copy(den_h.at[1], tmp_v)

  def den_body(j, carry):
    den_v[pl.ds(j * 16, 16)] = (den_v[pl.ds(j * 16, 16)]
                                + tmp_v[pl.ds(j * 16, 16)])
    return carry
  lax.fori_loop(0, NP // 16, den_body, 0)

  zf16 = jnp.zeros((16,), jnp.float32)
  for r in range(16):
    for q in range(HID // 16):
      zrow_v[r, pl.ds(q * 16, 16)] = zf16

  def zacc_body(i, carry):
    pltpu.sync_copy(zrow_v, acc_sh.at[pl.ds(s * 640 + i * 16, 16)])
    return carry
  lax.fori_loop(0, 40, zacc_body, 0)
  plsc.subcore_barrier()

  def edge_body(i, carry):
    off = i * 16
    s16 = src_v[pl.ds(off, 16)]
    d16 = dst_v[pl.ds(off, 16)]
    ex16 = ex_v[pl.ds(off, 16)]
    den16 = plsc.load_gather(den_v, [d16])
    w16 = ex16 / (den16 + 1e-16)
    wb_v[...] = w16
    pltpu.async_copy(h_h.at[s16], rows_v, sem).wait()
    for k in range(16):
      kk = jnp.full((16,), k, jnp.int32)
      wbk = plsc.load_gather(wb_v, [kk])
      for q in range(HID // 16):
        wrows_v[k, pl.ds(q * 16, 16)] = rows_v[k, pl.ds(q * 16, 16)] * wbk
    pltpu.sync_copy(wrows_v, acc_sh.at[d16], add=True)
    return carry
  lax.fori_loop(0, NBLK, edge_body, 0)

  plsc.subcore_barrier()
  pltpu.sync_copy(acc_sh.at[pl.ds(s * 640, 640)],
                  out_h.at[c, pl.ds(s * 640, 640)])


_sc_agg = pl.kernel(
    _sc_agg_body,
    out_type=jax.ShapeDtypeStruct((2, NP, HID), jnp.float32),
    mesh=_MESH,
    scratch_types=[
        pltpu.VMEM((EPT,), jnp.int32),
        pltpu.VMEM((EPT,), jnp.int32),
        pltpu.VMEM((EPT,), jnp.float32),
        pltpu.VMEM((NP,), jnp.float32),
        pltpu.VMEM((NP,), jnp.float32),
        pltpu.VMEM((16, HID), jnp.float32),
        pltpu.VMEM((16, HID), jnp.float32),
        pltpu.VMEM((16,), jnp.float32),
        pltpu.VMEM((16, HID), jnp.float32),
        pltpu.VMEM_SHARED((NP, HID), jnp.float32),
        pltpu.SemaphoreType.DMA,
    ],
)


def kernel(x, edge_index, batch, W1, a_src1, a_dst1, b1,
           W2, a_src2, a_dst2, b2, fcW, fcb):
  src = edge_index[0].astype(jnp.int32)
  dst = edge_index[1].astype(jnp.int32)
  loops = jnp.arange(N, dtype=jnp.int32)
  padz = jnp.zeros((EP - ET,), jnp.int32)
  src_p = jnp.concatenate([src, loops, padz])
  dst_p = jnp.concatenate([dst, loops, padz])
  x_p = jnp.pad(x, ((0, NP - N), (0, 0)))
  batch2d = jnp.pad(batch.astype(jnp.int32), (0, NP - N),
                    constant_values=NUM_GRAPHS).reshape(1, NP)
  aa1 = jnp.stack([a_src1, a_dst1], axis=1)
  aa2 = jnp.stack([a_src2, a_dst2], axis=1)

  h1, al1 = _tc_prologue(x_p, W1, aa1)
  ex1, den1 = _sc_attn(src_p, dst_p, al1)
  agg1 = _sc_agg(src_p, dst_p, ex1, den1, h1)
  h2, al2 = _tc_mid(agg1, b1.reshape(1, HID), W2, aa2)
  ex2, den2 = _sc_attn(src_p, dst_p, ex1 if False else al2)
  agg2 = _sc_agg(src_p, dst_p, ex2, den2, h2)
  return _tc_final(agg2, b2.reshape(1, HID), batch2d, fcW,
                   fcb.reshape(1, LAT))


# speedup vs baseline: 15.7727x; 15.7727x over previous
import jax
import jax.numpy as jnp
from jax import lax
from jax.experimental import pallas as pl
from jax.experimental.pallas import tpu as pltpu
from jax.experimental.pallas import tpu_sc as plsc

N = 10000
NP = 10240
F_IN = 128
HID = 128
LAT = 64
NUM_GRAPHS = 64
NEG_SLOPE = 0.2

E = 320000
ET = E + N
EP = 330240
NW = 32
EPT = EP // NW
CH = NP // 16
NSLAB = 5
SLAB = EPT // NSLAB

_MESH = plsc.VectorSubcoreMesh(
    core_axis_name="c", subcore_axis_name="s", num_cores=2, num_subcores=16)

_GDN = jax.lax.GatherDimensionNumbers(
    offset_dims=(), collapsed_slice_dims=(0,), start_index_map=(0,))


def _bcast_lane(vec, k):
  return jax.lax.gather(
      vec, jnp.full((16, 1), k, jnp.int32), dimension_numbers=_GDN,
      slice_sizes=(1,), mode=jax.lax.GatherScatterMode.PROMISE_IN_BOUNDS)


def _dot(a, b):
  return jax.lax.dot_general(
      a, b, (((a.ndim - 1,), (0,)), ((), ())),
      precision=jax.lax.Precision.HIGHEST,
      preferred_element_type=jnp.float32)


def _tc_prologue_body(x_ref, w_ref, aa_ref, h_ref, al_ref):
  h = _dot(x_ref[...], w_ref[...])
  h_ref[...] = h
  al_ref[...] = _dot(h, aa_ref[...])


def _tc_mid_body(agg_ref, b_ref, w_ref, aa_ref, h_ref, al_ref):
  xin = jax.nn.relu(agg_ref[0] + agg_ref[1] + b_ref[...])
  h = _dot(xin, w_ref[...])
  h_ref[...] = h
  al_ref[...] = _dot(h, aa_ref[...])


def _tc_final_body(agg_ref, b_ref, batch_ref, fcw_ref, fcb_ref, z_ref):
  hf = jax.nn.relu(agg_ref[0] + agg_ref[1] + b_ref[...])
  ids = jnp.broadcast_to(batch_ref[...], (NUM_GRAPHS, NP))
  gids = lax.broadcasted_iota(jnp.int32, (NUM_GRAPHS, NP), 0)
  p = jnp.where(ids == gids, 1.0, 0.0).astype(jnp.float32)
  sums = _dot(p, hf)
  counts = jnp.sum(p, axis=1, keepdims=True)
  pooled = sums / jnp.maximum(counts, 1.0)
  z_ref[...] = _dot(pooled, fcw_ref[...]) + fcb_ref[...]


def _tc_prologue(x, w, aa):
  return pl.pallas_call(
      _tc_prologue_body,
      out_shape=[jax.ShapeDtypeStruct((NP, HID), jnp.float32),
                 jax.ShapeDtypeStruct((NP, 2), jnp.float32)],
  )(x, w, aa)


def _tc_mid(agg, b, w, aa):
  return pl.pallas_call(
      _tc_mid_body,
      out_shape=[jax.ShapeDtypeStruct((NP, HID), jnp.float32),
                 jax.ShapeDtypeStruct((NP, 2), jnp.float32)],
  )(agg, b, w, aa)


def _tc_final(agg, b, batch2d, fcw, fcb):
  return pl.pallas_call(
      _tc_final_body,
      out_shape=jax.ShapeDtypeStruct((NUM_GRAPHS, LAT), jnp.float32),
  )(agg, b, batch2d, fcw, fcb)


def _sc_attn_body(src_h, dst_h, asrc_h, adst_h, ex_h, den_h,
                  src_v, dst_v, asrc_v, adst_v, ex_v, den_v, red_v, tmp_v,
                  den_sh, sem):
  c = lax.axis_index("c")
  s = lax.axis_index("s")
  wid = c * 16 + s
  base = wid * EPT

  pltpu.sync_copy(src_h.at[pl.ds(base, EPT)], src_v)
  pltpu.sync_copy(dst_h.at[pl.ds(base, EPT)], dst_v)
  pltpu.sync_copy(asrc_h, asrc_v)
  pltpu.sync_copy(adst_h, adst_v)

  zf16 = jnp.zeros((16,), jnp.float32)

  def zero_body(i, carry):
    den_v[pl.ds(i * 16, 16)] = zf16
    return carry
  lax.fori_loop(0, NP // 16, zero_body, 0)

  def edge_body(i, carry):
    off = i * 16
    s16 = src_v[pl.ds(off, 16)]
    d16 = dst_v[pl.ds(off, 16)]
    a_s = plsc.load_gather(asrc_v, [s16])
    a_d = plsc.load_gather(adst_v, [d16])
    e = a_s + a_d
    e = jnp.where(e >= 0.0, e, NEG_SLOPE * e)
    ex = jnp.exp(e)
    ex_v[pl.ds(off, 16)] = ex
    plsc.addupdate_scatter(den_v, [d16], ex)
    return carry
  lax.fori_loop(0, EPT // 16, edge_body, 0)

  pltpu.sync_copy(ex_v, ex_h.at[pl.ds(base, EPT)])

  pltpu.sync_copy(den_v, den_sh.at[s])
  plsc.subcore_barrier()
  cbase = s * CH
  pltpu.sync_copy(den_sh.at[0, pl.ds(cbase, CH)], red_v)
  for t in range(1, 16):
    pltpu.sync_copy(den_sh.at[t, pl.ds(cbase, CH)], tmp_v)

    def add_body(j, carry):
      red_v[pl.ds(j * 16, 16)] = (red_v[pl.ds(j * 16, 16)]
                                  + tmp_v[pl.ds(j * 16, 16)])
      return carry
    lax.fori_loop(0, CH // 16, add_body, 0)
  pltpu.sync_copy(red_v, den_h.at[c, pl.ds(cbase, CH)])


_sc_attn = pl.kernel(
    _sc_attn_body,
    out_type=[jax.ShapeDtypeStruct((EP,), jnp.float32),
              jax.ShapeDtypeStruct((2, NP), jnp.float32)],
    mesh=_MESH,
    compiler_params=pltpu.CompilerParams(needs_layout_passes=False),
    scratch_types=[
        pltpu.VMEM((EPT,), jnp.int32),
        pltpu.VMEM((EPT,), jnp.int32),
        pltpu.VMEM((NP,), jnp.float32),
        pltpu.VMEM((NP,), jnp.float32),
        pltpu.VMEM((EPT,), jnp.float32),
        pltpu.VMEM((NP,), jnp.float32),
        pltpu.VMEM((CH,), jnp.float32),
        pltpu.VMEM((CH,), jnp.float32),
        pltpu.VMEM_SHARED((16, NP), jnp.float32),
        pltpu.SemaphoreType.DMA,
    ],
)


def _sc_agg_body(src_h, dst_h, ex_h, den_h, h_h, out_h,
                 src_v, dst_v, ex_v, den_v, tmp_v, rows_v, wrows_v,
                 zrow_v, acc_sh, sem):
  c = lax.axis_index("c")
  s = lax.axis_index("s")
  wid = c * 16 + s
  base = wid * EPT

  pltpu.sync_copy(den_h.at[0], den_v)
  pltpu.sync_copy(den_h.at[1], tmp_v)

  def den_body(j, carry):
    den_v[pl.ds(j * 16, 16)] = (den_v[pl.ds(j * 16, 16)]
                                + tmp_v[pl.ds(j * 16, 16)])
    return carry
  lax.fori_loop(0, NP // 16, den_body, 0)

  zf16 = jnp.zeros((16,), jnp.float32)
  for r in range(16):
    for q in range(HID // 16):
      zrow_v[r, pl.ds(q * 16, 16)] = zf16

  def zacc_body(i, carry):
    pltpu.sync_copy(zrow_v, acc_sh.at[pl.ds(s * 640 + i * 16, 16)])
    return carry
  lax.fori_loop(0, 40, zacc_body, 0)
  plsc.subcore_barrier()

  def edge_body(i, carry):
    off = i * 16
    s16 = src_v[pl.ds(off, 16)]
    d16 = dst_v[pl.ds(off, 16)]
    ex16 = ex_v[pl.ds(off, 16)]
    den16 = plsc.load_gather(den_v, [d16])
    w16 = ex16 / (den16 + 1e-16)
    pltpu.async_copy(h_h.at[s16], rows_v, sem).wait()
    for k in range(16):
      wbk = _bcast_lane(w16, k)
      for q in range(HID // 16):
        wrows_v[k, pl.ds(q * 16, 16)] = rows_v[k, pl.ds(q * 16, 16)] * wbk
    pltpu.sync_copy(wrows_v, acc_sh.at[d16], add=True)
    return carry

  for slab in range(NSLAB):
    sbase = base + slab * SLAB
    pltpu.sync_copy(src_h.at[pl.ds(sbase, SLAB)], src_v)
    pltpu.sync_copy(dst_h.at[pl.ds(sbase, SLAB)], dst_v)
    pltpu.sync_copy(ex_h.at[pl.ds(sbase, SLAB)], ex_v)
    lax.fori_loop(0, SLAB // 16, edge_body, 0)

  plsc.subcore_barrier()
  pltpu.sync_copy(acc_sh.at[pl.ds(s * 640, 640)],
                  out_h.at[c, pl.ds(s * 640, 640)])


_sc_agg = pl.kernel(
    _sc_agg_body,
    out_type=jax.ShapeDtypeStruct((2, NP, HID), jnp.float32),
    mesh=_MESH,
    compiler_params=pltpu.CompilerParams(needs_layout_passes=False),
    scratch_types=[
        pltpu.VMEM((SLAB,), jnp.int32),
        pltpu.VMEM((SLAB,), jnp.int32),
        pltpu.VMEM((SLAB,), jnp.float32),
        pltpu.VMEM((NP,), jnp.float32),
        pltpu.VMEM((NP,), jnp.float32),
        pltpu.VMEM((16, HID), jnp.float32),
        pltpu.VMEM((16, HID), jnp.float32),
        pltpu.VMEM((16, HID), jnp.float32),
        pltpu.VMEM_SHARED((NP, HID), jnp.float32),
        pltpu.SemaphoreType.DMA,
    ],
)


def kernel(x, edge_index, batch, W1, a_src1, a_dst1, b1,
           W2, a_src2, a_dst2, b2, fcW, fcb):
  src = edge_index[0].astype(jnp.int32)
  dst = edge_index[1].astype(jnp.int32)
  loops = jnp.arange(N, dtype=jnp.int32)
  padn = jnp.full((EP - ET,), N, jnp.int32)
  src_p = jnp.concatenate([src, loops, padn])
  dst_p = jnp.concatenate([dst, loops, padn])
  x_p = jnp.pad(x, ((0, NP - N), (0, 0)))
  batch2d = jnp.pad(batch.astype(jnp.int32), (0, NP - N),
                    constant_values=NUM_GRAPHS).reshape(1, NP)
  aa1 = jnp.stack([a_src1, a_dst1], axis=1)
  aa2 = jnp.stack([a_src2, a_dst2], axis=1)

  h1, al1 = _tc_prologue(x_p, W1, aa1)
  ex1, den1 = _sc_attn(src_p, dst_p, al1[:, 0], al1[:, 1])
  agg1 = _sc_agg(src_p, dst_p, ex1, den1, h1)
  h2, al2 = _tc_mid(agg1, b1.reshape(1, HID), W2, aa2)
  ex2, den2 = _sc_attn(src_p, dst_p, al2[:, 0], al2[:, 1])
  agg2 = _sc_agg(src_p, dst_p, ex2, den2, h2)
  return _tc_final(agg2, b2.reshape(1, HID), batch2d, fcW,
                   fcb.reshape(1, LAT))


# speedup vs baseline: 25.7950x; 1.6354x over previous
import jax
import jax.numpy as jnp
from jax import lax
from jax.experimental import pallas as pl
from jax.experimental.pallas import tpu as pltpu
from jax.experimental.pallas import tpu_sc as plsc

N = 10000
NP = 10240
F_IN = 128
HID = 128
LAT = 64
NUM_GRAPHS = 64
NEG_SLOPE = 0.2

E = 320000
ET = E + N
EP = 331776
NW = 32
EPT = EP // NW
CH = NP // 16
NSLAB = 3
SLAB = EPT // NSLAB
GB = 64

_MESH = plsc.VectorSubcoreMesh(
    core_axis_name="c", subcore_axis_name="s", num_cores=2, num_subcores=16)

_GDN = jax.lax.GatherDimensionNumbers(
    offset_dims=(), collapsed_slice_dims=(0,), start_index_map=(0,))


def _bcast_lane(vec, k):
  return jax.lax.gather(
      vec, jnp.full((16, 1), k, jnp.int32), dimension_numbers=_GDN,
      slice_sizes=(1,), mode=jax.lax.GatherScatterMode.PROMISE_IN_BOUNDS)


def _dot(a, b):
  return jax.lax.dot_general(
      a, b, (((a.ndim - 1,), (0,)), ((), ())),
      precision=jax.lax.Precision.HIGHEST,
      preferred_element_type=jnp.float32)


def _tc_prologue_body(x_ref, w_ref, aa_ref, h_ref, al_ref):
  h = _dot(x_ref[...], w_ref[...])
  h_ref[...] = h
  al_ref[...] = _dot(h, aa_ref[...])


def _tc_mid_body(agg_ref, b_ref, w_ref, aa_ref, h_ref, al_ref):
  xin = jax.nn.relu(agg_ref[0] + agg_ref[1] + b_ref[...])
  h = _dot(xin, w_ref[...])
  h_ref[...] = h
  al_ref[...] = _dot(h, aa_ref[...])


def _tc_final_body(agg_ref, b_ref, batch_ref, fcw_ref, fcb_ref, z_ref):
  hf = jax.nn.relu(agg_ref[0] + agg_ref[1] + b_ref[...])
  ids = jnp.broadcast_to(batch_ref[...], (NUM_GRAPHS, NP))
  gids = lax.broadcasted_iota(jnp.int32, (NUM_GRAPHS, NP), 0)
  p = jnp.where(ids == gids, 1.0, 0.0).astype(jnp.float32)
  sums = _dot(p, hf)
  counts = jnp.sum(p, axis=1, keepdims=True)
  pooled = sums / jnp.maximum(counts, 1.0)
  z_ref[...] = _dot(pooled, fcw_ref[...]) + fcb_ref[...]


def _tc_prologue(x, w, aa):
  return pl.pallas_call(
      _tc_prologue_body,
      out_shape=[jax.ShapeDtypeStruct((NP, HID), jnp.float32),
                 jax.ShapeDtypeStruct((NP, 2), jnp.float32)],
  )(x, w, aa)


def _tc_mid(agg, b, w, aa):
  return pl.pallas_call(
      _tc_mid_body,
      out_shape=[jax.ShapeDtypeStruct((NP, HID), jnp.float32),
                 jax.ShapeDtypeStruct((NP, 2), jnp.float32)],
  )(agg, b, w, aa)


def _tc_final(agg, b, batch2d, fcw, fcb):
  return pl.pallas_call(
      _tc_final_body,
      out_shape=jax.ShapeDtypeStruct((NUM_GRAPHS, LAT), jnp.float32),
  )(agg, b, batch2d, fcw, fcb)


def _sc_attn_body(src_h, dst_h, asrc_h, adst_h, ex_h, den_h,
                  src_v, dst_v, asrc_v, adst_v, ex_v, den_v, red_v, tmp_v,
                  den_sh, sem):
  c = lax.axis_index("c")
  s = lax.axis_index("s")
  wid = c * 16 + s
  base = wid * EPT

  pltpu.sync_copy(src_h.at[pl.ds(base, EPT)], src_v)
  pltpu.sync_copy(dst_h.at[pl.ds(base, EPT)], dst_v)
  pltpu.sync_copy(asrc_h, asrc_v)
  pltpu.sync_copy(adst_h, adst_v)

  zf16 = jnp.zeros((16,), jnp.float32)

  def zero_body(i, carry):
    den_v[pl.ds(i * 16, 16)] = zf16
    return carry
  lax.fori_loop(0, NP // 16, zero_body, 0)

  def edge_body(i, carry):
    off = i * 16
    s16 = src_v[pl.ds(off, 16)]
    d16 = dst_v[pl.ds(off, 16)]
    a_s = plsc.load_gather(asrc_v, [s16])
    a_d = plsc.load_gather(adst_v, [d16])
    e = a_s + a_d
    e = jnp.where(e >= 0.0, e, NEG_SLOPE * e)
    ex = jnp.exp(e)
    ex_v[pl.ds(off, 16)] = ex
    plsc.addupdate_scatter(den_v, [d16], ex)
    return carry
  lax.fori_loop(0, EPT // 16, edge_body, 0)

  pltpu.sync_copy(ex_v, ex_h.at[pl.ds(base, EPT)])

  pltpu.sync_copy(den_v, den_sh.at[s])
  plsc.subcore_barrier()
  cbase = s * CH
  pltpu.sync_copy(den_sh.at[0, pl.ds(cbase, CH)], red_v)
  for t in range(1, 16):
    pltpu.sync_copy(den_sh.at[t, pl.ds(cbase, CH)], tmp_v)

    def add_body(j, carry):
      red_v[pl.ds(j * 16, 16)] = (red_v[pl.ds(j * 16, 16)]
                                  + tmp_v[pl.ds(j * 16, 16)])
      return carry
    lax.fori_loop(0, CH // 16, add_body, 0)
  pltpu.sync_copy(red_v, den_h.at[c, pl.ds(cbase, CH)])


_sc_attn = pl.kernel(
    _sc_attn_body,
    out_type=[jax.ShapeDtypeStruct((EP,), jnp.float32),
              jax.ShapeDtypeStruct((2, NP), jnp.float32)],
    mesh=_MESH,
    compiler_params=pltpu.CompilerParams(needs_layout_passes=False),
    scratch_types=[
        pltpu.VMEM((EPT,), jnp.int32),
        pltpu.VMEM((EPT,), jnp.int32),
        pltpu.VMEM((NP,), jnp.float32),
        pltpu.VMEM((NP,), jnp.float32),
        pltpu.VMEM((EPT,), jnp.float32),
        pltpu.VMEM((NP,), jnp.float32),
        pltpu.VMEM((CH,), jnp.float32),
        pltpu.VMEM((CH,), jnp.float32),
        pltpu.VMEM_SHARED((16, NP), jnp.float32),
        pltpu.SemaphoreType.DMA,
    ],
)


def _sc_agg_body(src_h, dst_h, ex_h, den_h, h_h, out_h,
                 src_v, dst_v, ex_v, den_v, tmp_v, rows_v, wrows_v,
                 acc_sh, sem, sem2):
  c = lax.axis_index("c")
  s = lax.axis_index("s")
  wid = c * 16 + s
  base = wid * EPT

  pltpu.sync_copy(den_h.at[0], den_v)
  pltpu.sync_copy(den_h.at[1], tmp_v)

  def den_body(j, carry):
    den_v[pl.ds(j * 16, 16)] = (den_v[pl.ds(j * 16, 16)]
                                + tmp_v[pl.ds(j * 16, 16)])
    return carry
  lax.fori_loop(0, NP // 16, den_body, 0)

  zf16 = jnp.zeros((16,), jnp.float32)
  for r in range(16):
    for q in range(HID // 16):
      wrows_v[r, pl.ds(q * 16, 16)] = zf16

  def zacc_body(i, carry):
    pltpu.sync_copy(wrows_v.at[pl.ds(0, 16)],
                    acc_sh.at[pl.ds(s * 640 + i * 16, 16)])
    return carry
  lax.fori_loop(0, 40, zacc_body, 0)
  plsc.subcore_barrier()

  def edge_body(i, carry):
    off = i * GB
    pltpu.async_copy(h_h.at[src_v.at[pl.ds(off, GB)]], rows_v, sem).wait()
    descs = []
    for sb in range(GB // 16):
      o2 = off + sb * 16
      d16 = dst_v[pl.ds(o2, 16)]
      ex16 = ex_v[pl.ds(o2, 16)]
      den16 = plsc.load_gather(den_v, [d16])
      w16 = ex16 / (den16 + 1e-16)
      for k in range(16):
        wbk = _bcast_lane(w16, k)
        r = sb * 16 + k
        for q in range(HID // 16):
          wrows_v[r, pl.ds(q * 16, 16)] = rows_v[r, pl.ds(q * 16, 16)] * wbk
      descs.append(pltpu.async_copy(wrows_v.at[pl.ds(sb * 16, 16)],
                                    acc_sh.at[d16], sem2, add=True))
    for dsc in descs:
      dsc.wait()
    return carry

  for slab in range(NSLAB):
    sbase = base + slab * SLAB
    pltpu.sync_copy(src_h.at[pl.ds(sbase, SLAB)], src_v)
    pltpu.sync_copy(dst_h.at[pl.ds(sbase, SLAB)], dst_v)
    pltpu.sync_copy(ex_h.at[pl.ds(sbase, SLAB)], ex_v)
    lax.fori_loop(0, SLAB // GB, edge_body, 0)

  plsc.subcore_barrier()
  pltpu.sync_copy(acc_sh.at[pl.ds(s * 640, 640)],
                  out_h.at[c, pl.ds(s * 640, 640)])


_sc_agg = pl.kernel(
    _sc_agg_body,
    out_type=jax.ShapeDtypeStruct((2, NP, HID), jnp.float32),
    mesh=_MESH,
    compiler_params=pltpu.CompilerParams(needs_layout_passes=False),
    scratch_types=[
        pltpu.VMEM((SLAB,), jnp.int32),
        pltpu.VMEM((SLAB,), jnp.int32),
        pltpu.VMEM((SLAB,), jnp.float32),
        pltpu.VMEM((NP,), jnp.float32),
        pltpu.VMEM((NP,), jnp.float32),
        pltpu.VMEM((GB, HID), jnp.float32),
        pltpu.VMEM((GB, HID), jnp.float32),
        pltpu.VMEM_SHARED((NP, HID), jnp.float32),
        pltpu.SemaphoreType.DMA,
        pltpu.SemaphoreType.DMA,
    ],
)


def kernel(x, edge_index, batch, W1, a_src1, a_dst1, b1,
           W2, a_src2, a_dst2, b2, fcW, fcb):
  src = edge_index[0].astype(jnp.int32)
  dst = edge_index[1].astype(jnp.int32)
  loops = jnp.arange(N, dtype=jnp.int32)
  padn = jnp.full((EP - ET,), N, jnp.int32)
  src_p = jnp.concatenate([src, loops, padn])
  dst_p = jnp.concatenate([dst, loops, padn])
  x_p = jnp.pad(x, ((0, NP - N), (0, 0)))
  batch2d = jnp.pad(batch.astype(jnp.int32), (0, NP - N),
                    constant_values=NUM_GRAPHS).reshape(1, NP)
  aa1 = jnp.stack([a_src1, a_dst1], axis=1)
  aa2 = jnp.stack([a_src2, a_dst2], axis=1)

  h1, al1 = _tc_prologue(x_p, W1, aa1)
  ex1, den1 = _sc_attn(src_p, dst_p, al1[:, 0], al1[:, 1])
  agg1 = _sc_agg(src_p, dst_p, ex1, den1, h1)
  h2, al2 = _tc_mid(agg1, b1.reshape(1, HID), W2, aa2)
  ex2, den2 = _sc_attn(src_p, dst_p, al2[:, 0], al2[:, 1])
  agg2 = _sc_agg(src_p, dst_p, ex2, den2, h2)
  return _tc_final(agg2, b2.reshape(1, HID), batch2d, fcW,
                   fcb.reshape(1, LAT))
